# baseline (speedup 1.0000x reference)
import jax
import jax.numpy as jnp
from jax.experimental import pallas as pl
from jax.experimental.pallas import tpu as pltpu

_BI = 256


def _sage_block(adj_ref, h_ref, wt_ref, out_ref):
    a = jnp.minimum(adj_ref[...], 1.0).astype(jnp.bfloat16)
    x = jnp.dot(a, h_ref[...], preferred_element_type=jnp.float32)
    out_ref[...] = jnp.dot(x.astype(jnp.bfloat16), wt_ref[...],
                           preferred_element_type=jnp.float32)


def kernel(h, adj, W):
    n, d_in = h.shape
    d_out = W.shape[0]
    h16 = h.astype(jnp.bfloat16)
    wt16 = W.T.astype(jnp.bfloat16)
    grid = (n // _BI,)
    return pl.pallas_call(
        _sage_block,
        grid=grid,
        in_specs=[
            pl.BlockSpec((_BI, n), lambda i: (i, 0)),
            pl.BlockSpec((n, d_in), lambda i: (0, 0)),
            pl.BlockSpec((d_in, d_out), lambda i: (0, 0)),
        ],
        out_specs=pl.BlockSpec((_BI, d_out), lambda i: (i, 0)),
        out_shape=jax.ShapeDtypeStruct((n, d_out), jnp.float32),
        compiler_params=pltpu.CompilerParams(
            dimension_semantics=("arbitrary",),
        ),
    )(adj, h16, wt16)


# speedup vs baseline: 1.1046x; 1.1046x over previous
import jax
import jax.numpy as jnp
from jax.experimental import pallas as pl
from jax.experimental.pallas import tpu as pltpu

_BI = 256


def _sage_block(adj_ref, h_ref, wt_ref, out_ref):
    a = jnp.minimum(adj_ref[...], 1.0)
    x = jnp.dot(a, h_ref[...], preferred_element_type=jnp.float32)
    out_ref[...] = jnp.dot(x, wt_ref[...], preferred_element_type=jnp.float32)


def kernel(h, adj, W):
    n, d_in = h.shape
    d_out = W.shape[0]
    wt = W.T
    grid = (n // _BI,)
    return pl.pallas_call(
        _sage_block,
        grid=grid,
        in_specs=[
            pl.BlockSpec((_BI, n), lambda i: (i, 0)),
            pl.BlockSpec((n, d_in), lambda i: (0, 0)),
            pl.BlockSpec((d_in, d_out), lambda i: (0, 0)),
        ],
        out_specs=pl.BlockSpec((_BI, d_out), lambda i: (i, 0)),
        out_shape=jax.ShapeDtypeStruct((n, d_out), jnp.float32),
        compiler_params=pltpu.CompilerParams(
            dimension_semantics=("arbitrary",),
        ),
    )(adj, h, wt)


# speedup vs baseline: 1.2495x; 1.1312x over previous
import jax
import jax.numpy as jnp
from jax.experimental import pallas as pl
from jax.experimental.pallas import tpu as pltpu

_BI = 512


def _sage_block(adj_ref, h_ref, wt_ref, out_ref):
    a = jnp.minimum(adj_ref[...], 1.0)
    x = jnp.dot(a, h_ref[...], preferred_element_type=jnp.float32)
    out_ref[...] = jnp.dot(x, wt_ref[...], preferred_element_type=jnp.float32)


def kernel(h, adj, W):
    n, d_in = h.shape
    d_out = W.shape[0]
    wt = W.T
    grid = (n // _BI,)
    return pl.pallas_call(
        _sage_block,
        grid=grid,
        in_specs=[
            pl.BlockSpec((_BI, n), lambda i: (i, 0)),
            pl.BlockSpec((n, d_in), lambda i: (0, 0)),
            pl.BlockSpec((d_in, d_out), lambda i: (0, 0)),
        ],
        out_specs=pl.BlockSpec((_BI, d_out), lambda i: (i, 0)),
        out_shape=jax.ShapeDtypeStruct((n, d_out), jnp.float32),
        compiler_params=pltpu.CompilerParams(
            dimension_semantics=("arbitrary",),
        ),
    )(adj, h, wt)


# speedup vs baseline: 1.2666x; 1.0136x over previous
import jax
import jax.numpy as jnp
from jax.experimental import pallas as pl
from jax.experimental.pallas import tpu as pltpu

_BI = 1024


def _sage_block(adj_ref, h_ref, wt_ref, out_ref):
    a = jnp.minimum(adj_ref[...], 1.0)
    x = jnp.dot(a, h_ref[...], preferred_element_type=jnp.float32)
    out_ref[...] = jnp.dot(x, wt_ref[...], preferred_element_type=jnp.float32)


def kernel(h, adj, W):
    n, d_in = h.shape
    d_out = W.shape[0]
    wt = W.T
    grid = (n // _BI,)
    return pl.pallas_call(
        _sage_block,
        grid=grid,
        in_specs=[
            pl.BlockSpec((_BI, n), lambda i: (i, 0)),
            pl.BlockSpec((n, d_in), lambda i: (0, 0)),
            pl.BlockSpec((d_in, d_out), lambda i: (0, 0)),
        ],
        out_specs=pl.BlockSpec((_BI, d_out), lambda i: (i, 0)),
        out_shape=jax.ShapeDtypeStruct((n, d_out), jnp.float32),
        compiler_params=pltpu.CompilerParams(
            dimension_semantics=("arbitrary",),
        ),
    )(adj, h, wt)
